# baseline (speedup 1.0000x reference)
import functools

import jax
import jax.numpy as jnp
from jax import lax
from jax.experimental import pallas as pl
from jax.experimental.pallas import tpu as pltpu
from jax.experimental.pallas import tpu_sc as plsc

N = 10000
E = 320000
D = 128
A = 5
NEG = -1e30

K_SL = 5
E_SL = E // K_SL

_BLK_N = 10000


def _node_body(x_ref, wn_ref, bn_ref, an_ref, ws_ref, bs_ref, wd_ref, bd_ref,
               px_ref, ts_ref, td_ref):
    x = x_ref[...]
    s = lax.dot_general(x, wn_ref[...], (((1,), (1,)), ((), ())),
                        preferred_element_type=jnp.float32) + bn_ref[...]
    m = jnp.max(s, axis=1, keepdims=True)
    e = jnp.exp(s - m)
    w = e / jnp.sum(e, axis=1, keepdims=True)
    px = x + lax.dot_general(w, an_ref[...], (((1,), (0,)), ((), ())),
                             preferred_element_type=jnp.float32)
    px_ref[...] = px
    ts_ref[...] = lax.dot_general(px, ws_ref[...], (((1,), (1,)), ((), ())),
                                  preferred_element_type=jnp.float32) + bs_ref[...]
    td_ref[...] = lax.dot_general(px, wd_ref[...], (((1,), (1,)), ((), ())),
                                  preferred_element_type=jnp.float32) + bd_ref[...]


def _node_stage(x, wn8, bn8, an8, ws16, bs16, wd16, bd16):
    full = lambda shape: pl.BlockSpec(shape, lambda i: (0, 0))
    return pl.pallas_call(
        _node_body,
        grid=(N // _BLK_N,),
        in_specs=[
            pl.BlockSpec((_BLK_N, D), lambda i: (i, 0)),
            full((8, D)), full((1, 8)), full((8, D)),
            full((16, D)), full((1, 16)), full((16, D)), full((1, 16)),
        ],
        out_specs=[
            pl.BlockSpec((_BLK_N, D), lambda i: (i, 0)),
            pl.BlockSpec((_BLK_N, 16), lambda i: (i, 0)),
            pl.BlockSpec((_BLK_N, 16), lambda i: (i, 0)),
        ],
        out_shape=[
            jax.ShapeDtypeStruct((N, D), jnp.float32),
            jax.ShapeDtypeStruct((N, 16), jnp.float32),
            jax.ShapeDtypeStruct((N, 16), jnp.float32),
        ],
    )(x, wn8, bn8, an8, ws16, bs16, wd16, bd16)


_NC = 2
_NS = 16
_NW = _NC * _NS
_PER_W = E_SL // _NW
_CHUNK = 1000
_NCHUNK = _PER_W // _CHUNK


def _sc_gather_body(ts_hbm, td_hbm, si_hbm, di_hbm, gsum_out,
                    si_v, di_v, sa_v, sb_v, da_v, db_v, sem0, sem1):
    wid = lax.axis_index("s") * _NC + lax.axis_index("c")
    base = wid * _PER_W
    pltpu.sync_copy(si_hbm.at[pl.ds(base, _PER_W)], si_v)
    pltpu.sync_copy(di_hbm.at[pl.ds(base, _PER_W)], di_v)

    sbufs = (sa_v, sb_v)
    dbufs = (da_v, db_v)
    sems = (sem0, sem1)

    def start(c):
        off = c * _CHUNK
        p = sems[c % 2]
        cs = pltpu.async_copy(ts_hbm.at[si_v.at[pl.ds(off, _CHUNK)]],
                              sbufs[c % 2], p)
        cd = pltpu.async_copy(td_hbm.at[di_v.at[pl.ds(off, _CHUNK)]],
                              dbufs[c % 2], p)
        return (cs, cd)

    copies = [start(0)]
    for c in range(_NCHUNK):
        if c + 1 < _NCHUNK:
            copies.append(start(c + 1))
        cs, cd = copies[c]
        cs.wait()
        cd.wait()
        sb, db = sbufs[c % 2], dbufs[c % 2]

        def add_row(i, _):
            sb[i, :] = sb[i, :] + db[i, :]
            return _

        lax.fori_loop(0, _CHUNK, add_row, 0, unroll=8)
        pltpu.sync_copy(sb, gsum_out.at[pl.ds(base + c * _CHUNK, _CHUNK)])


@functools.cache
def _get_sc_gather():
    return functools.partial(
        pl.kernel,
        mesh=plsc.VectorSubcoreMesh(core_axis_name="c", subcore_axis_name="s"),
        out_type=jax.ShapeDtypeStruct((E_SL, 16), jnp.float32),
        scratch_types=[
            pltpu.VMEM((_PER_W,), jnp.int32),
            pltpu.VMEM((_PER_W,), jnp.int32),
            pltpu.VMEM((_CHUNK, 16), jnp.float32),
            pltpu.VMEM((_CHUNK, 16), jnp.float32),
            pltpu.VMEM((_CHUNK, 16), jnp.float32),
            pltpu.VMEM((_CHUNK, 16), jnp.float32),
            pltpu.SemaphoreType.DMA,
            pltpu.SemaphoreType.DMA,
        ],
        compiler_params=pltpu.CompilerParams(use_tc_tiling_on_sc=False),
    )(_sc_gather_body)


_BLK_E = 16000
_BLK_P = _BLK_E // 8
_NBLK_SL = E_SL // _BLK_E


def _edge_body(ep_ref, gsum_ref, a2_ref, out_ref):
    del ep_ref
    s = gsum_ref[...]
    s = jnp.where(s >= 0.0, s, 0.01 * s)
    m = jnp.max(s, axis=1, keepdims=True)
    e = jnp.exp(s - m)
    gj = lax.broadcasted_iota(jnp.int32, (128, 128), 0) // 16
    gk = lax.broadcasted_iota(jnp.int32, (128, 128), 1) // 16
    ones_bd = jnp.where(gj == gk, 1.0, 0.0).astype(jnp.float32)
    denom = lax.dot_general(e, ones_bd, (((1,), (0,)), ((), ())),
                            preferred_element_type=jnp.float32)
    b = e / denom
    ex = jnp.reshape(lax.broadcast_in_dim(b, (_BLK_P, 8, 128), (0, 2)),
                     (_BLK_E, 128))
    row8 = lax.broadcasted_iota(jnp.int32, (_BLK_E, 128), 0) % 8
    lane16 = lax.broadcasted_iota(jnp.int32, (_BLK_E, 128), 1) // 16
    bm = jnp.where(row8 == lane16, ex, 0.0)
    out_ref[...] = lax.dot_general(bm, a2_ref[...], (((1,), (0,)), ((), ())),
                                   preferred_element_type=jnp.float32)


def _edge_slice(ep, gsump, a2, k):
    return pl.pallas_call(
        _edge_body,
        grid=(_NBLK_SL,),
        in_specs=[
            pl.BlockSpec(memory_space=pl.ANY),
            pl.BlockSpec((_BLK_P, 128), lambda i: (i, 0)),
            pl.BlockSpec((128, D), lambda i: (0, 0)),
        ],
        out_specs=pl.BlockSpec((_BLK_E, D), lambda i, k=k: (i + k * _NBLK_SL, 0)),
        out_shape=jax.ShapeDtypeStruct((E, D), jnp.float32),
        input_output_aliases={0: 0},
        compiler_params=pltpu.CompilerParams(
            dimension_semantics=("arbitrary",)),
    )(ep, gsump, a2)


def _edge_seed(gsump, a2):
    return pl.pallas_call(
        _edge_body,
        grid=(_NBLK_SL,),
        in_specs=[
            pl.BlockSpec((8, 128), lambda i: (0, 0)),
            pl.BlockSpec((_BLK_P, 128), lambda i: (i, 0)),
            pl.BlockSpec((128, D), lambda i: (0, 0)),
        ],
        out_specs=pl.BlockSpec((_BLK_E, D), lambda i: (i, 0)),
        out_shape=jax.ShapeDtypeStruct((E, D), jnp.float32),
        compiler_params=pltpu.CompilerParams(
            dimension_semantics=("arbitrary",)),
    )(jnp.zeros((8, 128), jnp.float32), gsump, a2)


def kernel(x, edge_index, node_anchor, node_att_W, node_att_b,
           edge_anchor, edge_W, edge_b):
    f32 = jnp.float32
    wn8 = jnp.pad(node_att_W, ((0, 3), (0, 0)))
    bn8 = jnp.pad(node_att_b, (0, 3), constant_values=NEG).reshape(1, 8).astype(f32)
    an8 = jnp.pad(node_anchor, ((0, 3), (0, 0)))
    ws16 = jnp.pad(edge_W[:, :D], ((0, 11), (0, 0)))
    wd16 = jnp.pad(edge_W[:, D:], ((0, 11), (0, 0)))
    bs16 = jnp.pad(edge_b, (0, 11), constant_values=NEG).reshape(1, 16).astype(f32)
    bd16 = jnp.zeros((1, 16), f32)
    an16 = jnp.pad(edge_anchor, ((0, 11), (0, 0)))
    a2 = jnp.tile(an16, (8, 1))

    px, ts, td = _node_stage(x, wn8, bn8, an8, ws16, bs16, wd16, bd16)

    sc = _get_sc_gather()
    g = []
    for k in range(K_SL):
        sl = slice(k * E_SL, (k + 1) * E_SL)
        g.append(sc(ts, td, edge_index[0, sl], edge_index[1, sl]))

    ep = None
    for k in range(K_SL):
        gsump = jnp.reshape(g[k], (E_SL // 8, 128))
        if k == 0:
            ep = _edge_seed(gsump, a2)
        else:
            ep = _edge_slice(ep, gsump, a2, k)
    return (px, ep)


# speedup vs baseline: 1.1703x; 1.1703x over previous
import functools

import jax
import jax.numpy as jnp
from jax import lax
from jax.experimental import pallas as pl
from jax.experimental.pallas import tpu as pltpu
from jax.experimental.pallas import tpu_sc as plsc

N = 10000
E = 320000
D = 128
A = 5
NEG = -1e30

K_SL = 5
E_SL = E // K_SL

_BLK_N = 10000


def _node_body(x_ref, wn_ref, bn_ref, an_ref, ws_ref, bs_ref, wd_ref, bd_ref,
               px_ref, ts_ref, td_ref):
    x = x_ref[...]
    s = lax.dot_general(x, wn_ref[...], (((1,), (1,)), ((), ())),
                        preferred_element_type=jnp.float32) + bn_ref[...]
    m = jnp.max(s, axis=1, keepdims=True)
    e = jnp.exp(s - m)
    w = e / jnp.sum(e, axis=1, keepdims=True)
    px = x + lax.dot_general(w, an_ref[...], (((1,), (0,)), ((), ())),
                             preferred_element_type=jnp.float32)
    px_ref[...] = px
    ts_ref[...] = lax.dot_general(px, ws_ref[...], (((1,), (1,)), ((), ())),
                                  preferred_element_type=jnp.float32) + bs_ref[...]
    td_ref[...] = lax.dot_general(px, wd_ref[...], (((1,), (1,)), ((), ())),
                                  preferred_element_type=jnp.float32) + bd_ref[...]


def _node_stage(x, wn8, bn8, an8, ws16, bs16, wd16, bd16):
    full = lambda shape: pl.BlockSpec(shape, lambda i: (0, 0))
    return pl.pallas_call(
        _node_body,
        grid=(N // _BLK_N,),
        in_specs=[
            pl.BlockSpec((_BLK_N, D), lambda i: (i, 0)),
            full((8, D)), full((1, 8)), full((8, D)),
            full((16, D)), full((1, 16)), full((16, D)), full((1, 16)),
        ],
        out_specs=[
            pl.BlockSpec((_BLK_N, D), lambda i: (i, 0)),
            pl.BlockSpec((_BLK_N, 16), lambda i: (i, 0)),
            pl.BlockSpec((_BLK_N, 16), lambda i: (i, 0)),
        ],
        out_shape=[
            jax.ShapeDtypeStruct((N, D), jnp.float32),
            jax.ShapeDtypeStruct((N, 16), jnp.float32),
            jax.ShapeDtypeStruct((N, 16), jnp.float32),
        ],
    )(x, wn8, bn8, an8, ws16, bs16, wd16, bd16)


_NC = 2
_NS = 16
_NW = _NC * _NS
_PER_W = E_SL // _NW
_CHUNK = 1000
_NCHUNK = _PER_W // _CHUNK


def _sc_gather_body(ts_hbm, td_hbm, si_hbm, di_hbm, gs_out, gd_out,
                    si_v, di_v, rows0_v, rows1_v, sem0, sem1):
    wid = lax.axis_index("s") * _NC + lax.axis_index("c")
    base = wid * _PER_W
    pltpu.sync_copy(si_hbm.at[pl.ds(base, _PER_W)], si_v)
    pltpu.sync_copy(di_hbm.at[pl.ds(base, _PER_W)], di_v)

    bufs = (rows0_v, rows1_v)
    sems = (sem0, sem1)
    def start(g):
        tab, idx = (ts_hbm, si_v) if g < _NCHUNK else (td_hbm, di_v)
        c = (g % _NCHUNK) * _CHUNK
        return pltpu.async_copy(tab.at[idx.at[pl.ds(c, _CHUNK)]],
                                bufs[g % 2], sems[g % 2])

    copies = [start(0)]
    for g in range(2 * _NCHUNK):
        if g + 1 < 2 * _NCHUNK:
            copies.append(start(g + 1))
        copies[g].wait()
        out = gs_out if g < _NCHUNK else gd_out
        off = base + (g % _NCHUNK) * _CHUNK
        pltpu.sync_copy(bufs[g % 2], out.at[pl.ds(off, _CHUNK)])


@functools.cache
def _get_sc_gather():
    return functools.partial(
        pl.kernel,
        mesh=plsc.VectorSubcoreMesh(core_axis_name="c", subcore_axis_name="s"),
        out_type=[
            jax.ShapeDtypeStruct((E_SL, 16), jnp.float32),
            jax.ShapeDtypeStruct((E_SL, 16), jnp.float32),
        ],
        scratch_types=[
            pltpu.VMEM((_PER_W,), jnp.int32),
            pltpu.VMEM((_PER_W,), jnp.int32),
            pltpu.VMEM((_CHUNK, 16), jnp.float32),
            pltpu.VMEM((_CHUNK, 16), jnp.float32),
            pltpu.SemaphoreType.DMA,
            pltpu.SemaphoreType.DMA,
        ],
        compiler_params=pltpu.CompilerParams(use_tc_tiling_on_sc=False),
    )(_sc_gather_body)


_BLK_E = 32000
_BLK_P = _BLK_E // 8
_NBLK_SL = E_SL // _BLK_E


def _edge_body(ep_ref, gs_ref, gd_ref, a2_ref, out_ref):
    del ep_ref
    s = gs_ref[...] + gd_ref[...]
    s = jnp.where(s >= 0.0, s, 0.01 * s)
    m = jnp.max(s, axis=1, keepdims=True)
    e = jnp.exp(s - m)
    gj = lax.broadcasted_iota(jnp.int32, (128, 128), 0) // 16
    gk = lax.broadcasted_iota(jnp.int32, (128, 128), 1) // 16
    ones_bd = jnp.where(gj == gk, 1.0, 0.0).astype(jnp.float32)
    denom = lax.dot_general(e, ones_bd, (((1,), (0,)), ((), ())),
                            preferred_element_type=jnp.float32)
    b = e / denom
    ex = jnp.reshape(lax.broadcast_in_dim(b, (_BLK_P, 8, 128), (0, 2)),
                     (_BLK_E, 128))
    row8 = lax.broadcasted_iota(jnp.int32, (_BLK_E, 128), 0) % 8
    lane16 = lax.broadcasted_iota(jnp.int32, (_BLK_E, 128), 1) // 16
    bm = jnp.where(row8 == lane16, ex, 0.0)
    out_ref[...] = lax.dot_general(bm, a2_ref[...], (((1,), (0,)), ((), ())),
                                   preferred_element_type=jnp.float32)


def _edge_slice(ep, gsp, gdp, a2, k):
    return pl.pallas_call(
        _edge_body,
        grid=(_NBLK_SL,),
        in_specs=[
            pl.BlockSpec(memory_space=pl.ANY),
            pl.BlockSpec((_BLK_P, 128), lambda i: (i, 0)),
            pl.BlockSpec((_BLK_P, 128), lambda i: (i, 0)),
            pl.BlockSpec((128, D), lambda i: (0, 0)),
        ],
        out_specs=pl.BlockSpec((_BLK_E, D), lambda i, k=k: (i + k * _NBLK_SL, 0)),
        out_shape=jax.ShapeDtypeStruct((E, D), jnp.float32),
        input_output_aliases={0: 0},
        compiler_params=pltpu.CompilerParams(
            dimension_semantics=("arbitrary",)),
    )(ep, gsp, gdp, a2)


def _edge_seed(gsp, gdp, a2):
    return pl.pallas_call(
        _edge_body,
        grid=(_NBLK_SL,),
        in_specs=[
            pl.BlockSpec((8, 128), lambda i: (0, 0)),
            pl.BlockSpec((_BLK_P, 128), lambda i: (i, 0)),
            pl.BlockSpec((_BLK_P, 128), lambda i: (i, 0)),
            pl.BlockSpec((128, D), lambda i: (0, 0)),
        ],
        out_specs=pl.BlockSpec((_BLK_E, D), lambda i: (i, 0)),
        out_shape=jax.ShapeDtypeStruct((E, D), jnp.float32),
        compiler_params=pltpu.CompilerParams(
            dimension_semantics=("arbitrary",)),
    )(jnp.zeros((8, 128), jnp.float32), gsp, gdp, a2)


def kernel(x, edge_index, node_anchor, node_att_W, node_att_b,
           edge_anchor, edge_W, edge_b):
    f32 = jnp.float32
    wn8 = jnp.pad(node_att_W, ((0, 3), (0, 0)))
    bn8 = jnp.pad(node_att_b, (0, 3), constant_values=NEG).reshape(1, 8).astype(f32)
    an8 = jnp.pad(node_anchor, ((0, 3), (0, 0)))
    ws16 = jnp.pad(edge_W[:, :D], ((0, 11), (0, 0)))
    wd16 = jnp.pad(edge_W[:, D:], ((0, 11), (0, 0)))
    bs16 = jnp.pad(edge_b, (0, 11), constant_values=NEG).reshape(1, 16).astype(f32)
    bd16 = jnp.zeros((1, 16), f32)
    an16 = jnp.pad(edge_anchor, ((0, 11), (0, 0)))
    a2 = jnp.tile(an16, (8, 1))

    px, ts, td = _node_stage(x, wn8, bn8, an8, ws16, bs16, wd16, bd16)

    sc = _get_sc_gather()
    g = []
    for k in range(K_SL):
        sl = slice(k * E_SL, (k + 1) * E_SL)
        g.append(sc(ts, td, edge_index[0, sl], edge_index[1, sl]))

    ep = None
    for k in range(K_SL):
        gsp = jnp.reshape(g[k][0], (E_SL // 8, 128))
        gdp = jnp.reshape(g[k][1], (E_SL // 8, 128))
        if k == 0:
            ep = _edge_seed(gsp, gdp, a2)
        else:
            ep = _edge_slice(ep, gsp, gdp, a2, k)
    return (px, ep)


# speedup vs baseline: 1.1706x; 1.0002x over previous
import functools

import jax
import jax.numpy as jnp
from jax import lax
from jax.experimental import pallas as pl
from jax.experimental.pallas import tpu as pltpu
from jax.experimental.pallas import tpu_sc as plsc

N = 10000
E = 320000
D = 128
A = 5
NEG = -1e30

K_SL = 2
E_SL = E // K_SL

_BLK_N = 10000


def _node_body(x_ref, wn_ref, bn_ref, an_ref, ws_ref, bs_ref, wd_ref, bd_ref,
               px_ref, ts_ref, td_ref):
    x = x_ref[...]
    s = lax.dot_general(x, wn_ref[...], (((1,), (1,)), ((), ())),
                        preferred_element_type=jnp.float32) + bn_ref[...]
    m = jnp.max(s, axis=1, keepdims=True)
    e = jnp.exp(s - m)
    w = e / jnp.sum(e, axis=1, keepdims=True)
    px = x + lax.dot_general(w, an_ref[...], (((1,), (0,)), ((), ())),
                             preferred_element_type=jnp.float32)
    px_ref[...] = px
    ts_ref[...] = lax.dot_general(px, ws_ref[...], (((1,), (1,)), ((), ())),
                                  preferred_element_type=jnp.float32) + bs_ref[...]
    td_ref[...] = lax.dot_general(px, wd_ref[...], (((1,), (1,)), ((), ())),
                                  preferred_element_type=jnp.float32) + bd_ref[...]


def _node_stage(x, wn8, bn8, an8, ws16, bs16, wd16, bd16):
    full = lambda shape: pl.BlockSpec(shape, lambda i: (0, 0))
    return pl.pallas_call(
        _node_body,
        grid=(N // _BLK_N,),
        in_specs=[
            pl.BlockSpec((_BLK_N, D), lambda i: (i, 0)),
            full((8, D)), full((1, 8)), full((8, D)),
            full((16, D)), full((1, 16)), full((16, D)), full((1, 16)),
        ],
        out_specs=[
            pl.BlockSpec((_BLK_N, D), lambda i: (i, 0)),
            pl.BlockSpec((_BLK_N, 16), lambda i: (i, 0)),
            pl.BlockSpec((_BLK_N, 16), lambda i: (i, 0)),
        ],
        out_shape=[
            jax.ShapeDtypeStruct((N, D), jnp.float32),
            jax.ShapeDtypeStruct((N, 16), jnp.float32),
            jax.ShapeDtypeStruct((N, 16), jnp.float32),
        ],
    )(x, wn8, bn8, an8, ws16, bs16, wd16, bd16)


_NC = 2
_NS = 16
_NW = _NC * _NS
_PER_W = E_SL // _NW
_CHUNK = 1000
_NCHUNK = _PER_W // _CHUNK


def _sc_gather_body(ts_hbm, td_hbm, si_hbm, di_hbm, gs_out, gd_out,
                    si_v, di_v, rows0_v, rows1_v, sem0, sem1):
    wid = lax.axis_index("s") * _NC + lax.axis_index("c")
    base = wid * _PER_W
    pltpu.sync_copy(si_hbm.at[pl.ds(base, _PER_W)], si_v)
    pltpu.sync_copy(di_hbm.at[pl.ds(base, _PER_W)], di_v)

    bufs = (rows0_v, rows1_v)
    sems = (sem0, sem1)
    def start(g):
        tab, idx = (ts_hbm, si_v) if g < _NCHUNK else (td_hbm, di_v)
        c = (g % _NCHUNK) * _CHUNK
        return pltpu.async_copy(tab.at[idx.at[pl.ds(c, _CHUNK)]],
                                bufs[g % 2], sems[g % 2])

    copies = [start(0)]
    for g in range(2 * _NCHUNK):
        if g + 1 < 2 * _NCHUNK:
            copies.append(start(g + 1))
        copies[g].wait()
        out = gs_out if g < _NCHUNK else gd_out
        off = base + (g % _NCHUNK) * _CHUNK
        pltpu.sync_copy(bufs[g % 2], out.at[pl.ds(off, _CHUNK)])


@functools.cache
def _get_sc_gather():
    return functools.partial(
        pl.kernel,
        mesh=plsc.VectorSubcoreMesh(core_axis_name="c", subcore_axis_name="s"),
        out_type=[
            jax.ShapeDtypeStruct((E_SL, 16), jnp.float32),
            jax.ShapeDtypeStruct((E_SL, 16), jnp.float32),
        ],
        scratch_types=[
            pltpu.VMEM((_PER_W,), jnp.int32),
            pltpu.VMEM((_PER_W,), jnp.int32),
            pltpu.VMEM((_CHUNK, 16), jnp.float32),
            pltpu.VMEM((_CHUNK, 16), jnp.float32),
            pltpu.SemaphoreType.DMA,
            pltpu.SemaphoreType.DMA,
        ],
        compiler_params=pltpu.CompilerParams(use_tc_tiling_on_sc=False),
    )(_sc_gather_body)


_BLK_E = 32000
_BLK_P = _BLK_E // 8
_NBLK_SL = E_SL // _BLK_E


def _edge_body(ep_ref, gs_ref, gd_ref, a2_ref, out_ref):
    del ep_ref
    s = gs_ref[...] + gd_ref[...]
    s = jnp.where(s >= 0.0, s, 0.01 * s)
    m = jnp.max(s, axis=1, keepdims=True)
    e = jnp.exp(s - m)
    gj = lax.broadcasted_iota(jnp.int32, (128, 128), 0) // 16
    gk = lax.broadcasted_iota(jnp.int32, (128, 128), 1) // 16
    ones_bd = jnp.where(gj == gk, 1.0, 0.0).astype(jnp.float32)
    denom = lax.dot_general(e, ones_bd, (((1,), (0,)), ((), ())),
                            preferred_element_type=jnp.float32)
    b = e / denom
    ex = jnp.reshape(lax.broadcast_in_dim(b, (_BLK_P, 8, 128), (0, 2)),
                     (_BLK_E, 128))
    row8 = lax.broadcasted_iota(jnp.int32, (_BLK_E, 128), 0) % 8
    lane16 = lax.broadcasted_iota(jnp.int32, (_BLK_E, 128), 1) // 16
    bm = jnp.where(row8 == lane16, ex, 0.0)
    out_ref[...] = lax.dot_general(bm, a2_ref[...], (((1,), (0,)), ((), ())),
                                   preferred_element_type=jnp.float32)


def _edge_slice(ep, gsp, gdp, a2, k):
    return pl.pallas_call(
        _edge_body,
        grid=(_NBLK_SL,),
        in_specs=[
            pl.BlockSpec(memory_space=pl.ANY),
            pl.BlockSpec((_BLK_P, 128), lambda i: (i, 0)),
            pl.BlockSpec((_BLK_P, 128), lambda i: (i, 0)),
            pl.BlockSpec((128, D), lambda i: (0, 0)),
        ],
        out_specs=pl.BlockSpec((_BLK_E, D), lambda i, k=k: (i + k * _NBLK_SL, 0)),
        out_shape=jax.ShapeDtypeStruct((E, D), jnp.float32),
        input_output_aliases={0: 0},
        compiler_params=pltpu.CompilerParams(
            dimension_semantics=("arbitrary",)),
    )(ep, gsp, gdp, a2)


def _edge_seed(gsp, gdp, a2):
    return pl.pallas_call(
        _edge_body,
        grid=(_NBLK_SL,),
        in_specs=[
            pl.BlockSpec((8, 128), lambda i: (0, 0)),
            pl.BlockSpec((_BLK_P, 128), lambda i: (i, 0)),
            pl.BlockSpec((_BLK_P, 128), lambda i: (i, 0)),
            pl.BlockSpec((128, D), lambda i: (0, 0)),
        ],
        out_specs=pl.BlockSpec((_BLK_E, D), lambda i: (i, 0)),
        out_shape=jax.ShapeDtypeStruct((E, D), jnp.float32),
        compiler_params=pltpu.CompilerParams(
            dimension_semantics=("arbitrary",)),
    )(jnp.zeros((8, 128), jnp.float32), gsp, gdp, a2)


def kernel(x, edge_index, node_anchor, node_att_W, node_att_b,
           edge_anchor, edge_W, edge_b):
    f32 = jnp.float32
    wn8 = jnp.pad(node_att_W, ((0, 3), (0, 0)))
    bn8 = jnp.pad(node_att_b, (0, 3), constant_values=NEG).reshape(1, 8).astype(f32)
    an8 = jnp.pad(node_anchor, ((0, 3), (0, 0)))
    ws16 = jnp.pad(edge_W[:, :D], ((0, 11), (0, 0)))
    wd16 = jnp.pad(edge_W[:, D:], ((0, 11), (0, 0)))
    bs16 = jnp.pad(edge_b, (0, 11), constant_values=NEG).reshape(1, 16).astype(f32)
    bd16 = jnp.zeros((1, 16), f32)
    an16 = jnp.pad(edge_anchor, ((0, 11), (0, 0)))
    a2 = jnp.tile(an16, (8, 1))

    px, ts, td = _node_stage(x, wn8, bn8, an8, ws16, bs16, wd16, bd16)

    sc = _get_sc_gather()
    g = []
    for k in range(K_SL):
        sl = slice(k * E_SL, (k + 1) * E_SL)
        g.append(sc(ts, td, edge_index[0, sl], edge_index[1, sl]))

    ep = None
    for k in range(K_SL):
        gsp = jnp.reshape(g[k][0], (E_SL // 8, 128))
        gdp = jnp.reshape(g[k][1], (E_SL // 8, 128))
        if k == 0:
            ep = _edge_seed(gsp, gdp, a2)
        else:
            ep = _edge_slice(ep, gsp, gdp, a2, k)
    return (px, ep)
